# Initial kernel scaffold; baseline (speedup 1.0000x reference)
import jax
import jax.numpy as jnp
from jax.experimental import pallas as pl


def kernel(inputs, W_node, b_node, W_router, b_router, W_leaf_node, b_leaf_node, W_leaf_dense, b_leaf_dense):
    raise NotImplementedError("write your pallas kernel here")


# speedup vs baseline: 1.6027x; 1.6027x over previous
import functools

import jax
import jax.numpy as jnp
from jax.experimental import pallas as pl
from jax.experimental.pallas import tpu as pltpu

_TM = 256


def _root_body(x_ref, wn_ref, bn_ref, wr_ref, br_ref, dec_ref, arg_ref):
    x = x_ref[...]
    h = jnp.maximum(
        jnp.dot(x, wn_ref[...], preferred_element_type=jnp.float32) + bn_ref[...],
        0.0,
    )
    logits = jnp.dot(h, wr_ref[...], preferred_element_type=jnp.float32) + br_ref[...]
    m = jnp.max(logits, axis=-1, keepdims=True)
    p = jnp.exp(logits - m)
    dec = p / jnp.sum(p, axis=-1, keepdims=True)
    dec_ref[...] = dec
    e = logits.shape[-1]
    idx = jax.lax.broadcasted_iota(jnp.int32, logits.shape, 1)
    mx = jnp.max(dec, axis=-1, keepdims=True)
    arg_ref[...] = jnp.min(jnp.where(dec == mx, idx, e), axis=-1, keepdims=True)


def _leaf_body(tile_ref, exp_ref, valid_ref, off_ref,
               xs_ref, wn_ref, bn_ref, wd_ref, bd_ref, out_ref):
    i = pl.program_id(0)
    t = tile_ref[i]
    prev_t = tile_ref[jnp.maximum(i - 1, 0)]
    first_visit = jnp.logical_or(i == 0, t != prev_t)

    @pl.when(first_visit)
    def _():
        out_ref[...] = jnp.zeros_like(out_ref)

    e = exp_ref[i]
    start = off_ref[e]
    end = off_ref[e + 1]
    tm = xs_ref.shape[0]
    rows = t * tm + jax.lax.broadcasted_iota(jnp.int32, (tm, 1), 0)
    mask = (rows >= start) & (rows < end) & (valid_ref[i] > 0)

    x = xs_ref[...]
    h = jnp.maximum(
        jnp.dot(x, wn_ref[0], preferred_element_type=jnp.float32) + bn_ref[0],
        0.0,
    )
    logits = jnp.dot(h, wd_ref[0], preferred_element_type=jnp.float32) + bd_ref[0]
    m = jnp.max(logits, axis=-1, keepdims=True)
    p = jnp.exp(logits - m)
    dec = p / jnp.sum(p, axis=-1, keepdims=True)
    out_ref[...] += jnp.where(mask, dec, 0.0)


def kernel(inputs, W_node, b_node, W_router, b_router,
           W_leaf_node, b_leaf_node, W_leaf_dense, b_leaf_dense):
    n, d = inputs.shape
    e = W_router.shape[1]
    nt = n // _TM
    max_items = nt + e - 1

    dec_out, arg2d = pl.pallas_call(
        _root_body,
        grid=(nt,),
        in_specs=[
            pl.BlockSpec((_TM, d), lambda i: (i, 0)),
            pl.BlockSpec((d, d), lambda i: (0, 0)),
            pl.BlockSpec((1, d), lambda i: (0, 0)),
            pl.BlockSpec((d, e), lambda i: (0, 0)),
            pl.BlockSpec((1, e), lambda i: (0, 0)),
        ],
        out_specs=[
            pl.BlockSpec((_TM, e), lambda i: (i, 0)),
            pl.BlockSpec((_TM, 1), lambda i: (i, 0)),
        ],
        out_shape=[
            jax.ShapeDtypeStruct((n, e), jnp.float32),
            jax.ShapeDtypeStruct((n, 1), jnp.int32),
        ],
        compiler_params=pltpu.CompilerParams(
            dimension_semantics=("arbitrary",),
        ),
    )(inputs, W_node, b_node.reshape(1, d), W_router, b_router.reshape(1, e))

    decision = arg2d[:, 0]
    perm = jnp.argsort(decision, stable=True)

    counts = jnp.sum(decision[:, None] == jnp.arange(e, dtype=jnp.int32)[None, :],
                     axis=0, dtype=jnp.int32)
    offsets = jnp.cumsum(counts)
    starts = offsets - counts
    first_tile = starts // _TM
    last_tile = jnp.where(counts > 0, (offsets - 1) // _TM, first_tile)
    tiles_touched = jnp.where(counts > 0, last_tile - first_tile + 1, 0)
    group_ids = jnp.repeat(jnp.arange(e, dtype=jnp.int32), tiles_touched,
                           total_repeat_length=max_items)
    tiles_before = jnp.cumsum(tiles_touched) - tiles_touched
    item_idx = jnp.arange(max_items, dtype=jnp.int32)
    raw_tile = first_tile[group_ids] + item_idx - tiles_before[group_ids]
    total_items = jnp.sum(tiles_touched)
    valid = (item_idx < total_items).astype(jnp.int32)
    tile_ids = jnp.clip(jnp.where(valid > 0, raw_tile, nt - 1), 0, nt - 1)
    tile_ids = tile_ids.astype(jnp.int32)
    off_full = jnp.concatenate([jnp.zeros((1,), jnp.int32), offsets]).astype(jnp.int32)

    xs = jnp.take(inputs, perm, axis=0)

    grid_spec = pltpu.PrefetchScalarGridSpec(
        num_scalar_prefetch=4,
        grid=(max_items,),
        in_specs=[
            pl.BlockSpec((_TM, d), lambda i, t, ex, v, o: (t[i], 0)),
            pl.BlockSpec((1, d, d), lambda i, t, ex, v, o: (ex[i], 0, 0)),
            pl.BlockSpec((1, 1, d), lambda i, t, ex, v, o: (ex[i], 0, 0)),
            pl.BlockSpec((1, d, e), lambda i, t, ex, v, o: (ex[i], 0, 0)),
            pl.BlockSpec((1, 1, e), lambda i, t, ex, v, o: (ex[i], 0, 0)),
        ],
        out_specs=pl.BlockSpec((_TM, e), lambda i, t, ex, v, o: (t[i], 0)),
    )
    sorted_dec = pl.pallas_call(
        _leaf_body,
        grid_spec=grid_spec,
        out_shape=jax.ShapeDtypeStruct((n, e), jnp.float32),
        compiler_params=pltpu.CompilerParams(
            dimension_semantics=("arbitrary",),
        ),
    )(tile_ids, group_ids, valid, off_full,
      xs, W_leaf_node, b_leaf_node.reshape(e, 1, d),
      W_leaf_dense, b_leaf_dense.reshape(e, 1, e))

    ordered = jnp.take(sorted_dec, perm, axis=0)
    return jnp.concatenate([dec_out[:, None, :], ordered[:, None, :]], axis=1)


# speedup vs baseline: 1.6855x; 1.0517x over previous
import functools

import jax
import jax.numpy as jnp
from jax import lax
from jax.experimental import pallas as pl
from jax.experimental.pallas import tpu as pltpu
from jax.experimental.pallas import tpu_sc as plsc

_TM = 256
_NW = 32
_CH = 64


def _root_body(x_ref, wn_ref, bn_ref, wr_ref, br_ref,
               dec_ref, arg_ref, rank_ref, counts_ref):
    i = pl.program_id(0)
    x = x_ref[...]
    h = jnp.maximum(
        jnp.dot(x, wn_ref[...], preferred_element_type=jnp.float32) + bn_ref[...],
        0.0,
    )
    logits = jnp.dot(h, wr_ref[...], preferred_element_type=jnp.float32) + br_ref[...]
    m = jnp.max(logits, axis=-1, keepdims=True)
    p = jnp.exp(logits - m)
    dec = p / jnp.sum(p, axis=-1, keepdims=True)
    dec_ref[...] = dec
    e = logits.shape[-1]
    idx = jax.lax.broadcasted_iota(jnp.int32, dec.shape, 1)
    mx = jnp.max(dec, axis=-1, keepdims=True)
    am = jnp.min(jnp.where(dec == mx, idx, e), axis=-1, keepdims=True)
    arg_ref[...] = am

    @pl.when(i == 0)
    def _():
        counts_ref[...] = jnp.zeros_like(counts_ref)

    tm = x.shape[0]
    oh = (idx == am).astype(jnp.float32)
    r = jax.lax.broadcasted_iota(jnp.int32, (tm, tm), 0)
    c = jax.lax.broadcasted_iota(jnp.int32, (tm, tm), 1)
    tril = (r > c).astype(jnp.float32)
    strictcum = jnp.dot(tril, oh, preferred_element_type=jnp.float32)
    rank_tile = counts_ref[...].astype(jnp.float32) + strictcum
    rank = jnp.sum(oh * rank_tile, axis=-1, keepdims=True)
    rank_ref[...] = rank.astype(jnp.int32)
    counts_ref[...] += jnp.sum(oh, axis=0, keepdims=True).astype(jnp.int32)


def _leaf_body(tile_ref, exp_ref, valid_ref, off_ref,
               xs_ref, wn_ref, bn_ref, wd_ref, bd_ref, out_ref):
    i = pl.program_id(0)
    t = tile_ref[i]
    prev_t = tile_ref[jnp.maximum(i - 1, 0)]
    first_visit = jnp.logical_or(i == 0, t != prev_t)

    @pl.when(first_visit)
    def _():
        out_ref[...] = jnp.zeros_like(out_ref)

    e = exp_ref[i]
    start = off_ref[e]
    end = off_ref[e + 1]
    tm = xs_ref.shape[0]
    rows = t * tm + jax.lax.broadcasted_iota(jnp.int32, (tm, 1), 0)
    mask = (rows >= start) & (rows < end) & (valid_ref[i] > 0)

    x = xs_ref[...]
    h = jnp.maximum(
        jnp.dot(x, wn_ref[0], preferred_element_type=jnp.float32) + bn_ref[0],
        0.0,
    )
    logits = jnp.dot(h, wd_ref[0], preferred_element_type=jnp.float32) + bd_ref[0]
    m = jnp.max(logits, axis=-1, keepdims=True)
    p = jnp.exp(logits - m)
    dec = p / jnp.sum(p, axis=-1, keepdims=True)
    tm_, e_ = dec.shape
    pad = jnp.zeros((tm_, 128 - e_), dtype=dec.dtype)
    out_ref[...] += jnp.where(mask, jnp.concatenate([dec, pad], axis=-1), 0.0)


def _sc_scatter(y, pos, chunk):
    n, d = y.shape
    bpw = n // _NW
    nch = bpw // chunk
    mesh = plsc.VectorSubcoreMesh(core_axis_name="c", subcore_axis_name="s")

    @functools.partial(
        pl.kernel,
        out_type=jax.ShapeDtypeStruct((n, d), y.dtype),
        mesh=mesh,
        scratch_types=[
            pltpu.VMEM((chunk,), jnp.int32),
            pltpu.VMEM((chunk, d), y.dtype),
            pltpu.SemaphoreType.DMA,
        ],
    )
    def k(y_hbm, pos_hbm, out_hbm, idx_v, rows_v, sem):
        wid = lax.axis_index("s") * 2 + lax.axis_index("c")
        base = wid * bpw
        for ci in range(nch):
            off = base + ci * chunk
            pltpu.sync_copy(pos_hbm.at[pl.ds(off, chunk)], idx_v)
            pltpu.sync_copy(y_hbm.at[pl.ds(off, chunk)], rows_v)
            pltpu.async_copy(rows_v, out_hbm.at[idx_v], sem).wait()

    return k(y, pos)


def kernel(inputs, W_node, b_node, W_router, b_router,
           W_leaf_node, b_leaf_node, W_leaf_dense, b_leaf_dense):
    n, d = inputs.shape
    e = W_router.shape[1]
    nt = n // _TM
    max_items = nt + e - 1

    dec_out, arg2d, rank2d, counts2d = pl.pallas_call(
        _root_body,
        grid=(nt,),
        in_specs=[
            pl.BlockSpec((_TM, d), lambda i: (i, 0)),
            pl.BlockSpec((d, d), lambda i: (0, 0)),
            pl.BlockSpec((1, d), lambda i: (0, 0)),
            pl.BlockSpec((d, e), lambda i: (0, 0)),
            pl.BlockSpec((1, e), lambda i: (0, 0)),
        ],
        out_specs=[
            pl.BlockSpec((_TM, e), lambda i: (i, 0)),
            pl.BlockSpec((_TM, 1), lambda i: (i, 0)),
            pl.BlockSpec((_TM, 1), lambda i: (i, 0)),
            pl.BlockSpec((1, e), lambda i: (0, 0)),
        ],
        out_shape=[
            jax.ShapeDtypeStruct((n, e), jnp.float32),
            jax.ShapeDtypeStruct((n, 1), jnp.int32),
            jax.ShapeDtypeStruct((n, 1), jnp.int32),
            jax.ShapeDtypeStruct((1, e), jnp.int32),
        ],
        compiler_params=pltpu.CompilerParams(
            dimension_semantics=("arbitrary",),
        ),
    )(inputs, W_node, b_node.reshape(1, d), W_router, b_router.reshape(1, e))

    decision = arg2d[:, 0]
    counts = counts2d[0]
    offsets = jnp.cumsum(counts)
    starts = offsets - counts
    pos = jnp.take(starts, decision) + rank2d[:, 0]

    first_tile = starts // _TM
    last_tile = jnp.where(counts > 0, (offsets - 1) // _TM, first_tile)
    tiles_touched = jnp.where(counts > 0, last_tile - first_tile + 1, 0)
    group_ids = jnp.repeat(jnp.arange(e, dtype=jnp.int32), tiles_touched,
                           total_repeat_length=max_items)
    tiles_before = jnp.cumsum(tiles_touched) - tiles_touched
    item_idx = jnp.arange(max_items, dtype=jnp.int32)
    raw_tile = first_tile[group_ids] + item_idx - tiles_before[group_ids]
    total_items = jnp.sum(tiles_touched)
    valid = (item_idx < total_items).astype(jnp.int32)
    tile_ids = jnp.clip(jnp.where(valid > 0, raw_tile, nt - 1), 0, nt - 1)
    tile_ids = tile_ids.astype(jnp.int32)
    off_full = jnp.concatenate([jnp.zeros((1,), jnp.int32), offsets]).astype(jnp.int32)

    xs = _sc_scatter(inputs, pos, _CH)

    grid_spec = pltpu.PrefetchScalarGridSpec(
        num_scalar_prefetch=4,
        grid=(max_items,),
        in_specs=[
            pl.BlockSpec((_TM, d), lambda i, t, ex, v, o: (t[i], 0)),
            pl.BlockSpec((1, d, d), lambda i, t, ex, v, o: (ex[i], 0, 0)),
            pl.BlockSpec((1, 1, d), lambda i, t, ex, v, o: (ex[i], 0, 0)),
            pl.BlockSpec((1, d, e), lambda i, t, ex, v, o: (ex[i], 0, 0)),
            pl.BlockSpec((1, 1, e), lambda i, t, ex, v, o: (ex[i], 0, 0)),
        ],
        out_specs=pl.BlockSpec((_TM, 128), lambda i, t, ex, v, o: (t[i], 0)),
    )
    sorted_dec = pl.pallas_call(
        _leaf_body,
        grid_spec=grid_spec,
        out_shape=jax.ShapeDtypeStruct((n, 128), jnp.float32),
        compiler_params=pltpu.CompilerParams(
            dimension_semantics=("arbitrary",),
        ),
    )(tile_ids, group_ids, valid, off_full,
      xs, W_leaf_node, b_leaf_node.reshape(e, 1, d),
      W_leaf_dense, b_leaf_dense.reshape(e, 1, e))

    ordered = _sc_scatter(sorted_dec, pos, n // _NW)[:, :e]
    return jnp.concatenate([dec_out[:, None, :], ordered[:, None, :]], axis=1)


# speedup vs baseline: 1.8077x; 1.0725x over previous
import functools

import jax
import jax.numpy as jnp
from jax import lax
from jax.experimental import pallas as pl
from jax.experimental.pallas import tpu as pltpu
from jax.experimental.pallas import tpu_sc as plsc

_TM = 256
_NW = 32
_CH = 64


def _root_body(x_ref, wn_ref, bn_ref, wr_ref, br_ref,
               out_ref, counts_ref, hist_ref):
    i = pl.program_id(0)
    h = jnp.maximum(
        jnp.dot(x_ref[...], wn_ref[...], preferred_element_type=jnp.float32)
        + bn_ref[...],
        0.0,
    )
    logits = (
        jnp.dot(h, wr_ref[...], preferred_element_type=jnp.float32) + br_ref[...]
    )
    out_ref[...] = logits
    e = logits.shape[-1]
    idx = jax.lax.broadcasted_iota(jnp.int32, logits.shape, 1)
    mx = jnp.max(logits, axis=-1, keepdims=True)
    am = jnp.min(jnp.where(logits == mx, idx, e), axis=-1, keepdims=True)
    oh = idx == am

    @pl.when(i == 0)
    def _():
        hist_ref[...] = jnp.zeros_like(hist_ref)

    hist_ref[...] += jnp.sum(oh.astype(jnp.int32), axis=0, keepdims=True)

    @pl.when(i == pl.num_programs(0) - 1)
    def _():
        counts_ref[...] = hist_ref[...]


def _epi_body(logits_ref, cnt_ref, dec_ref, pos_ref,
              carry_ref, starts_ref, tril_ref):
    i = pl.program_id(0)
    logits = logits_ref[...]
    tm, e = logits.shape
    idx = jax.lax.broadcasted_iota(jnp.int32, logits.shape, 1)
    mx = jnp.max(logits, axis=-1, keepdims=True)
    am = jnp.min(jnp.where(logits == mx, idx, e), axis=-1, keepdims=True)
    oh = idx == am

    @pl.when(i == 0)
    def _():
        carry_ref[...] = jnp.zeros_like(carry_ref)
        r8 = jax.lax.broadcasted_iota(jnp.int32, (e, e), 0)
        c8 = jax.lax.broadcasted_iota(jnp.int32, (e, e), 1)
        sut = (r8 < c8).astype(jnp.float32)
        starts_ref[...] = jnp.dot(cnt_ref[...].astype(jnp.float32), sut,
                                  preferred_element_type=jnp.float32)
        r = jax.lax.broadcasted_iota(jnp.int32, (tm, tm), 0)
        c = jax.lax.broadcasted_iota(jnp.int32, (tm, tm), 1)
        tril_ref[...] = (r > c).astype(jnp.bfloat16)

    p = jnp.exp(logits - mx)
    dec_ref[...] = p / jnp.sum(p, axis=-1, keepdims=True)
    ohf = oh.astype(jnp.float32)
    strictcum = jnp.dot(tril_ref[...], oh.astype(jnp.bfloat16),
                        preferred_element_type=jnp.float32)
    slot = starts_ref[...] + carry_ref[...] + strictcum
    pos_ref[...] = jnp.sum(ohf * slot, axis=-1, keepdims=True).astype(jnp.int32)
    carry_ref[...] += jnp.sum(ohf, axis=0, keepdims=True)


def _leaf_body(tile_ref, exp_ref, valid_ref, off_ref,
               xs_ref, wn_ref, bn_ref, wd_ref, bd_ref, out_ref):
    i = pl.program_id(0)
    t = tile_ref[i]
    prev_t = tile_ref[jnp.maximum(i - 1, 0)]
    first_visit = jnp.logical_or(i == 0, t != prev_t)

    @pl.when(first_visit)
    def _():
        out_ref[...] = jnp.zeros_like(out_ref)

    e = exp_ref[i]
    start = off_ref[e]
    end = off_ref[e + 1]
    tm = xs_ref.shape[0]
    rows = t * tm + jax.lax.broadcasted_iota(jnp.int32, (tm, 1), 0)
    mask = (rows >= start) & (rows < end) & (valid_ref[i] > 0)

    x = xs_ref[...]
    h = jnp.maximum(
        jnp.dot(x, wn_ref[0], preferred_element_type=jnp.float32) + bn_ref[0],
        0.0,
    )
    logits = jnp.dot(h, wd_ref[0], preferred_element_type=jnp.float32) + bd_ref[0]
    m = jnp.max(logits, axis=-1, keepdims=True)
    p = jnp.exp(logits - m)
    dec = p / jnp.sum(p, axis=-1, keepdims=True)
    tm_, e_ = dec.shape
    pad = jnp.zeros((tm_, 128 - e_), dtype=dec.dtype)
    out_ref[...] += jnp.where(mask, jnp.concatenate([dec, pad], axis=-1), 0.0)


def _sc_scatter(y, pos, chunk):
    n, d = y.shape
    bpw = n // _NW
    nch = bpw // chunk
    mesh = plsc.VectorSubcoreMesh(core_axis_name="c", subcore_axis_name="s")

    @functools.partial(
        pl.kernel,
        out_type=jax.ShapeDtypeStruct((n, d), y.dtype),
        mesh=mesh,
        scratch_types=[
            pltpu.VMEM((chunk,), jnp.int32),
            pltpu.VMEM((chunk, d), y.dtype),
            pltpu.SemaphoreType.DMA,
        ],
    )
    def k(y_hbm, pos_hbm, out_hbm, idx_v, rows_v, sem):
        wid = lax.axis_index("s") * 2 + lax.axis_index("c")
        base = wid * bpw
        for ci in range(nch):
            off = base + ci * chunk
            pltpu.sync_copy(pos_hbm.at[pl.ds(off, chunk)], idx_v)
            pltpu.sync_copy(y_hbm.at[pl.ds(off, chunk)], rows_v)
            pltpu.async_copy(rows_v, out_hbm.at[idx_v], sem).wait()

    return k(y, pos)


def kernel(inputs, W_node, b_node, W_router, b_router,
           W_leaf_node, b_leaf_node, W_leaf_dense, b_leaf_dense):
    n, d = inputs.shape
    e = W_router.shape[1]
    nt = n // _TM
    max_items = nt + e - 1

    logits, counts2d = pl.pallas_call(
        _root_body,
        grid=(nt,),
        in_specs=[
            pl.BlockSpec((_TM, d), lambda i: (i, 0)),
            pl.BlockSpec((d, d), lambda i: (0, 0)),
            pl.BlockSpec((1, d), lambda i: (0, 0)),
            pl.BlockSpec((d, e), lambda i: (0, 0)),
            pl.BlockSpec((1, e), lambda i: (0, 0)),
        ],
        out_specs=[
            pl.BlockSpec((_TM, e), lambda i: (i, 0)),
            pl.BlockSpec((1, e), lambda i: (0, 0)),
        ],
        out_shape=[
            jax.ShapeDtypeStruct((n, e), jnp.float32),
            jax.ShapeDtypeStruct((1, e), jnp.int32),
        ],
        scratch_shapes=[
            pltpu.VMEM((1, e), jnp.int32),
        ],
        compiler_params=pltpu.CompilerParams(
            dimension_semantics=("arbitrary",),
        ),
    )(inputs, W_node, b_node.reshape(1, d), W_router, b_router.reshape(1, e))

    tme = 1024
    dec_out, pos2d = pl.pallas_call(
        _epi_body,
        grid=(n // tme,),
        in_specs=[
            pl.BlockSpec((tme, e), lambda i: (i, 0)),
            pl.BlockSpec((1, e), lambda i: (0, 0)),
        ],
        out_specs=[
            pl.BlockSpec((tme, e), lambda i: (i, 0)),
            pl.BlockSpec((tme, 1), lambda i: (i, 0)),
        ],
        out_shape=[
            jax.ShapeDtypeStruct((n, e), jnp.float32),
            jax.ShapeDtypeStruct((n, 1), jnp.int32),
        ],
        scratch_shapes=[
            pltpu.VMEM((1, e), jnp.float32),
            pltpu.VMEM((1, e), jnp.float32),
            pltpu.VMEM((tme, tme), jnp.bfloat16),
        ],
        compiler_params=pltpu.CompilerParams(
            dimension_semantics=("arbitrary",),
        ),
    )(logits, counts2d)

    pos = pos2d[:, 0]
    counts = counts2d[0]
    offsets = jnp.cumsum(counts)
    starts = offsets - counts

    first_tile = starts // _TM
    last_tile = jnp.where(counts > 0, (offsets - 1) // _TM, first_tile)
    tiles_touched = jnp.where(counts > 0, last_tile - first_tile + 1, 0)
    group_ids = jnp.repeat(jnp.arange(e, dtype=jnp.int32), tiles_touched,
                           total_repeat_length=max_items)
    tiles_before = jnp.cumsum(tiles_touched) - tiles_touched
    item_idx = jnp.arange(max_items, dtype=jnp.int32)
    raw_tile = first_tile[group_ids] + item_idx - tiles_before[group_ids]
    total_items = jnp.sum(tiles_touched)
    valid = (item_idx < total_items).astype(jnp.int32)
    tile_ids = jnp.clip(jnp.where(valid > 0, raw_tile, nt - 1), 0, nt - 1)
    tile_ids = tile_ids.astype(jnp.int32)
    off_full = jnp.concatenate([jnp.zeros((1,), jnp.int32), offsets]).astype(jnp.int32)

    xs = _sc_scatter(inputs, pos, _CH)

    grid_spec = pltpu.PrefetchScalarGridSpec(
        num_scalar_prefetch=4,
        grid=(max_items,),
        in_specs=[
            pl.BlockSpec((_TM, d), lambda i, t, ex, v, o: (t[i], 0)),
            pl.BlockSpec((1, d, d), lambda i, t, ex, v, o: (ex[i], 0, 0)),
            pl.BlockSpec((1, 1, d), lambda i, t, ex, v, o: (ex[i], 0, 0)),
            pl.BlockSpec((1, d, e), lambda i, t, ex, v, o: (ex[i], 0, 0)),
            pl.BlockSpec((1, 1, e), lambda i, t, ex, v, o: (ex[i], 0, 0)),
        ],
        out_specs=pl.BlockSpec((_TM, 128), lambda i, t, ex, v, o: (t[i], 0)),
    )
    sorted_dec = pl.pallas_call(
        _leaf_body,
        grid_spec=grid_spec,
        out_shape=jax.ShapeDtypeStruct((n, 128), jnp.float32),
        compiler_params=pltpu.CompilerParams(
            dimension_semantics=("arbitrary",),
        ),
    )(tile_ids, group_ids, valid, off_full,
      xs, W_leaf_node, b_leaf_node.reshape(e, 1, d),
      W_leaf_dense, b_leaf_dense.reshape(e, 1, e))

    ordered = _sc_scatter(sorted_dec, pos, n // _NW)[:, :e]
    return jnp.concatenate([dec_out[:, None, :], ordered[:, None, :]], axis=1)


# speedup vs baseline: 1.8086x; 1.0005x over previous
import functools

import jax
import jax.numpy as jnp
from jax import lax
from jax.experimental import pallas as pl
from jax.experimental.pallas import tpu as pltpu
from jax.experimental.pallas import tpu_sc as plsc

_TM = 256
_NW = 32
_CH = 64


def _root_body(x_ref, wn_ref, bn_ref, wr_ref, br_ref,
               out_ref, counts_ref, hist_ref):
    i = pl.program_id(0)
    h = jnp.maximum(
        jnp.dot(x_ref[...], wn_ref[...], preferred_element_type=jnp.float32)
        + bn_ref[...],
        0.0,
    )
    logits = (
        jnp.dot(h, wr_ref[...], preferred_element_type=jnp.float32) + br_ref[...]
    )
    out_ref[...] = logits
    e = logits.shape[-1]
    idx = jax.lax.broadcasted_iota(jnp.int32, logits.shape, 1)
    mx = jnp.max(logits, axis=-1, keepdims=True)
    am = jnp.min(jnp.where(logits == mx, idx, e), axis=-1, keepdims=True)
    oh = idx == am

    @pl.when(i == 0)
    def _():
        hist_ref[...] = jnp.zeros_like(hist_ref)

    hist_ref[...] += jnp.sum(oh.astype(jnp.int32), axis=0, keepdims=True)

    @pl.when(i == pl.num_programs(0) - 1)
    def _():
        counts_ref[...] = hist_ref[...]


def _epi_body(logits_ref, cnt_ref, dec_ref, pos_ref,
              carry_ref, starts_ref, tril_ref):
    i = pl.program_id(0)
    logits = logits_ref[...]
    tm, e = logits.shape
    idx = jax.lax.broadcasted_iota(jnp.int32, logits.shape, 1)
    mx = jnp.max(logits, axis=-1, keepdims=True)
    am = jnp.min(jnp.where(logits == mx, idx, e), axis=-1, keepdims=True)
    oh = idx == am

    @pl.when(i == 0)
    def _():
        carry_ref[...] = jnp.zeros_like(carry_ref)
        lane = jax.lax.broadcasted_iota(jnp.int32, (1, e), 1)
        cnt = cnt_ref[...]
        starts = jnp.zeros_like(cnt)
        for j in range(1, e):
            sj = jnp.sum(jnp.where(lane < j, cnt, 0))
            starts = starts + jnp.where(lane == j, sj, 0)
        starts_ref[...] = starts.astype(jnp.float32)
        r = jax.lax.broadcasted_iota(jnp.int32, (tm, tm), 0)
        c = jax.lax.broadcasted_iota(jnp.int32, (tm, tm), 1)
        tril_ref[...] = (r > c).astype(jnp.bfloat16)

    p = jnp.exp(logits - mx)
    dec_ref[...] = p / jnp.sum(p, axis=-1, keepdims=True)
    ohf = oh.astype(jnp.float32)
    strictcum = jnp.dot(tril_ref[...], oh.astype(jnp.bfloat16),
                        preferred_element_type=jnp.float32)
    slot = starts_ref[...] + carry_ref[...] + strictcum
    pos_ref[...] = jnp.sum(ohf * slot, axis=-1, keepdims=True).astype(jnp.int32)
    carry_ref[...] += jnp.sum(ohf, axis=0, keepdims=True)


def _leaf_body(tile_ref, exp_ref, valid_ref, off_ref,
               xs_ref, wn_ref, bn_ref, wd_ref, bd_ref, out_ref):
    i = pl.program_id(0)
    t = tile_ref[i]
    prev_t = tile_ref[jnp.maximum(i - 1, 0)]
    first_visit = jnp.logical_or(i == 0, t != prev_t)

    @pl.when(first_visit)
    def _():
        out_ref[...] = jnp.zeros_like(out_ref)

    e = exp_ref[i]
    start = off_ref[e]
    end = off_ref[e + 1]
    tm = xs_ref.shape[0]
    rows = t * tm + jax.lax.broadcasted_iota(jnp.int32, (tm, 1), 0)
    mask = (rows >= start) & (rows < end) & (valid_ref[i] > 0)

    x = xs_ref[...]
    h = jnp.maximum(
        jnp.dot(x, wn_ref[0], preferred_element_type=jnp.float32) + bn_ref[0],
        0.0,
    )
    logits = jnp.dot(h, wd_ref[0], preferred_element_type=jnp.float32) + bd_ref[0]
    m = jnp.max(logits, axis=-1, keepdims=True)
    p = jnp.exp(logits - m)
    dec = p / jnp.sum(p, axis=-1, keepdims=True)
    tm_, e_ = dec.shape
    pad = jnp.zeros((tm_, 128 - e_), dtype=dec.dtype)
    out_ref[...] += jnp.where(mask, jnp.concatenate([dec, pad], axis=-1), 0.0)


def _sc_scatter(y, pos, chunk):
    n, d = y.shape
    bpw = n // _NW
    nch = bpw // chunk
    mesh = plsc.VectorSubcoreMesh(core_axis_name="c", subcore_axis_name="s")

    @functools.partial(
        pl.kernel,
        out_type=jax.ShapeDtypeStruct((n, d), y.dtype),
        mesh=mesh,
        scratch_types=[
            pltpu.VMEM((chunk,), jnp.int32),
            pltpu.VMEM((chunk, d), y.dtype),
            pltpu.SemaphoreType.DMA,
        ],
    )
    def k(y_hbm, pos_hbm, out_hbm, idx_v, rows_v, sem):
        wid = lax.axis_index("s") * 2 + lax.axis_index("c")
        base = wid * bpw
        for ci in range(nch):
            off = base + ci * chunk
            pltpu.sync_copy(pos_hbm.at[pl.ds(off, chunk)], idx_v)
            pltpu.sync_copy(y_hbm.at[pl.ds(off, chunk)], rows_v)
            pltpu.async_copy(rows_v, out_hbm.at[idx_v], sem).wait()

    return k(y, pos)


def _route(inputs, W_node, b_node, W_router, b_router):
    n, d = inputs.shape
    e = W_router.shape[1]
    nt = n // _TM

    logits, counts2d = pl.pallas_call(
        _root_body,
        grid=(nt,),
        in_specs=[
            pl.BlockSpec((_TM, d), lambda i: (i, 0)),
            pl.BlockSpec((d, d), lambda i: (0, 0)),
            pl.BlockSpec((1, d), lambda i: (0, 0)),
            pl.BlockSpec((d, e), lambda i: (0, 0)),
            pl.BlockSpec((1, e), lambda i: (0, 0)),
        ],
        out_specs=[
            pl.BlockSpec((_TM, e), lambda i: (i, 0)),
            pl.BlockSpec((1, e), lambda i: (0, 0)),
        ],
        out_shape=[
            jax.ShapeDtypeStruct((n, e), jnp.float32),
            jax.ShapeDtypeStruct((1, e), jnp.int32),
        ],
        scratch_shapes=[
            pltpu.VMEM((1, e), jnp.int32),
        ],
        compiler_params=pltpu.CompilerParams(
            dimension_semantics=("arbitrary",),
        ),
    )(inputs, W_node, b_node.reshape(1, d), W_router, b_router.reshape(1, e))

    tme = 1024
    dec_out, pos2d = pl.pallas_call(
        _epi_body,
        grid=(n // tme,),
        in_specs=[
            pl.BlockSpec((tme, e), lambda i: (i, 0)),
            pl.BlockSpec((1, e), lambda i: (0, 0)),
        ],
        out_specs=[
            pl.BlockSpec((tme, e), lambda i: (i, 0)),
            pl.BlockSpec((tme, 1), lambda i: (i, 0)),
        ],
        out_shape=[
            jax.ShapeDtypeStruct((n, e), jnp.float32),
            jax.ShapeDtypeStruct((n, 1), jnp.int32),
        ],
        scratch_shapes=[
            pltpu.VMEM((1, e), jnp.float32),
            pltpu.VMEM((1, e), jnp.float32),
            pltpu.VMEM((tme, tme), jnp.bfloat16),
        ],
        compiler_params=pltpu.CompilerParams(
            dimension_semantics=("arbitrary",),
        ),
    )(logits, counts2d)
    return dec_out, pos2d, counts2d


def kernel(inputs, W_node, b_node, W_router, b_router,
           W_leaf_node, b_leaf_node, W_leaf_dense, b_leaf_dense):
    n, d = inputs.shape
    e = W_router.shape[1]
    nt = n // _TM
    max_items = nt + e - 1

    dec_out, pos2d, counts2d = _route(inputs, W_node, b_node, W_router, b_router)

    pos = pos2d[:, 0]
    counts = counts2d[0]
    offsets = jnp.cumsum(counts)
    starts = offsets - counts

    first_tile = starts // _TM
    last_tile = jnp.where(counts > 0, (offsets - 1) // _TM, first_tile)
    tiles_touched = jnp.where(counts > 0, last_tile - first_tile + 1, 0)
    group_ids = jnp.repeat(jnp.arange(e, dtype=jnp.int32), tiles_touched,
                           total_repeat_length=max_items)
    tiles_before = jnp.cumsum(tiles_touched) - tiles_touched
    item_idx = jnp.arange(max_items, dtype=jnp.int32)
    raw_tile = first_tile[group_ids] + item_idx - tiles_before[group_ids]
    total_items = jnp.sum(tiles_touched)
    valid = (item_idx < total_items).astype(jnp.int32)
    tile_ids = jnp.clip(jnp.where(valid > 0, raw_tile, nt - 1), 0, nt - 1)
    tile_ids = tile_ids.astype(jnp.int32)
    off_full = jnp.concatenate([jnp.zeros((1,), jnp.int32), offsets]).astype(jnp.int32)

    xs = _sc_scatter(inputs, pos, _CH)

    grid_spec = pltpu.PrefetchScalarGridSpec(
        num_scalar_prefetch=4,
        grid=(max_items,),
        in_specs=[
            pl.BlockSpec((_TM, d), lambda i, t, ex, v, o: (t[i], 0)),
            pl.BlockSpec((1, d, d), lambda i, t, ex, v, o: (ex[i], 0, 0)),
            pl.BlockSpec((1, 1, d), lambda i, t, ex, v, o: (ex[i], 0, 0)),
            pl.BlockSpec((1, d, e), lambda i, t, ex, v, o: (ex[i], 0, 0)),
            pl.BlockSpec((1, 1, e), lambda i, t, ex, v, o: (ex[i], 0, 0)),
        ],
        out_specs=pl.BlockSpec((_TM, 128), lambda i, t, ex, v, o: (t[i], 0)),
    )
    sorted_dec = pl.pallas_call(
        _leaf_body,
        grid_spec=grid_spec,
        out_shape=jax.ShapeDtypeStruct((n, 128), jnp.float32),
        compiler_params=pltpu.CompilerParams(
            dimension_semantics=("arbitrary",),
        ),
    )(tile_ids, group_ids, valid, off_full,
      xs, W_leaf_node, b_leaf_node.reshape(e, 1, d),
      W_leaf_dense, b_leaf_dense.reshape(e, 1, e))

    ordered = _sc_scatter(sorted_dec, pos, n // _NW)[:, :e]
    return jnp.concatenate([dec_out[:, None, :], ordered[:, None, :]], axis=1)
